# baseline (speedup 1.0000x reference)
import functools
import math

import jax
import jax.numpy as jnp
from jax import lax
from jax.experimental import pallas as pl
from jax.experimental.pallas import tpu as pltpu
from jax.experimental.pallas import tpu_sc as plsc

D_MODEL = 64
SCALE = math.sqrt(D_MODEL)

_info = plsc.get_sparse_core_info()
NC, NS, L = _info.num_cores, _info.num_subcores, _info.num_lanes

SEQ = 200
BATCH = 4096
VOCAB = 1000000
ROWS_PER_TILE = 13
D_PER_CORE = D_MODEL // 2


def _make_kernel():
    mesh = plsc.VectorSubcoreMesh(core_axis_name="c", subcore_axis_name="s")
    n_idx = ROWS_PER_TILE * BATCH

    @functools.partial(
        pl.kernel,
        mesh=mesh,
        out_type=jax.ShapeDtypeStruct((SEQ, D_MODEL, BATCH), jnp.float32),
        scratch_types=[
            pltpu.VMEM((n_idx,), jnp.int32),
            pltpu.VMEM((3, BATCH), jnp.float32),
            pltpu.VMEM_SHARED((VOCAB,), jnp.float32),
            pltpu.SemaphoreType.DMA((3,)),
            pltpu.SemaphoreType.DMA((3,)),
        ],
        compiler_params=pltpu.CompilerParams(use_tc_tiling_on_sc=False),
    )
    def k(xT_hbm, lutT_hbm, out_hbm, x_v, bufs, sprow, gsem, osem):
        c = lax.axis_index("c")
        s = lax.axis_index("s")
        row0 = lax.min(ROWS_PER_TILE * s, SEQ - ROWS_PER_TILE)
        R = ROWS_PER_TILE

        pltpu.sync_copy(xT_hbm.at[pl.ds(row0 * BATCH, n_idx)], x_v)

        def one_round(r, carry):
            d = c * D_PER_CORE + r
            @pl.when(s < 8)
            def _():
                sl = pl.ds(s * (VOCAB // 8), VOCAB // 8)
                pltpu.sync_copy(lutT_hbm.at[d, sl], sprow.at[sl])

            plsc.subcore_barrier()

            def gather(j, issue):
                idx = x_v.at[pl.ds(j * BATCH, BATCH)]
                cp = pltpu.make_async_copy(sprow.at[idx], bufs.at[j % 3], gsem.at[j % 3])
                cp.start() if issue else cp.wait()

            def write(j, issue):
                cp = pltpu.make_async_copy(
                    bufs.at[j % 3], out_hbm.at[row0 + j, d], osem.at[j % 3]
                )
                cp.start() if issue else cp.wait()

            gather(0, True)
            gather(1, True)
            for j in range(R):
                if 1 <= j <= R - 3:
                    write(j - 1, False)
                if j <= R - 3:
                    gather(j + 2, True)
                gather(j, False)

                def scale(i, cc):
                    for u in range(4):
                        sl = pl.ds((i * 4 + u) * L, L)
                        bufs[j % 3, sl] = bufs[j % 3, sl] * SCALE
                    return cc

                lax.fori_loop(0, BATCH // (4 * L), scale, 0)
                write(j, True)
            for j in range(R - 3, R):
                write(j, False)
            plsc.subcore_barrier()
            return carry

        lax.fori_loop(0, D_PER_CORE, one_round, 0)

    return k


_kernel_fn = _make_kernel()


def kernel(x, lut):
    xT_flat = x.T.reshape(SEQ * BATCH)
    lutT = lut.T
    out = _kernel_fn(xT_flat, lutT)
    return out.transpose(2, 0, 1)


# speedup vs baseline: 6.3538x; 6.3538x over previous
import functools
import math

import jax
import jax.numpy as jnp
from jax import lax
from jax.experimental import pallas as pl
from jax.experimental.pallas import tpu as pltpu
from jax.experimental.pallas import tpu_sc as plsc

D_MODEL = 64
SCALE = math.sqrt(D_MODEL)

_info = plsc.get_sparse_core_info()
NC, NS, L = _info.num_cores, _info.num_subcores, _info.num_lanes
NW = NC * NS


def _make_kernel(B, D, C, NBUF, U):
    per_w = B // NW
    nchunks = per_w // C
    ngroups = nchunks // NBUF
    assert per_w % C == 0 and nchunks % NBUF == 0 and C % U == 0
    assert (C * D) % 128 == 0
    mesh = plsc.VectorSubcoreMesh(core_axis_name="c", subcore_axis_name="s")

    @functools.partial(
        pl.kernel,
        mesh=mesh,
        out_type=jax.ShapeDtypeStruct((B, 2 * D), jnp.float32),
        scratch_types=[
            pltpu.VMEM((NBUF, C), jnp.int32),
            pltpu.VMEM((NBUF, C, D), jnp.float32),
        ]
        + [pltpu.SemaphoreType.DMA] * (2 * NBUF),
        compiler_params=pltpu.CompilerParams(use_tc_tiling_on_sc=False),
    )
    def k(idx_hbm, lut_hbm, out_hbm, idx_v, rows_v, *sems):
        gsem, osem = sems[:NBUF], sems[NBUF:]
        wid = lax.axis_index("s") * NC + lax.axis_index("c")
        base = wid * per_w

        def scale_chunk(b):
            def body(r0, carry):
                for u in range(U):
                    r = r0 * U + u
                    for j in range(D // L):
                        sl = pl.ds(j * L, L)
                        rows_v[b, r, sl] = rows_v[b, r, sl] * SCALE
                return carry

            lax.fori_loop(0, C // U, body, 0)

        def out_slice(g):
            return out_hbm.at[pl.ds(base + g * C, C), pl.ds(0, D)]

        for b in range(NBUF):
            row0 = base + b * C
            pltpu.sync_copy(idx_hbm.at[pl.ds(row0, C)], idx_v.at[b])
            pltpu.async_copy(lut_hbm.at[idx_v.at[b]], rows_v.at[b], gsem[b])

        def group(gi, carry):
            for b in range(NBUF):
                g = gi * NBUF + b
                row0 = base + g * C
                pltpu.make_async_copy(
                    lut_hbm.at[idx_v.at[b]], rows_v.at[b], gsem[b]
                ).wait()
                scale_chunk(b)
                pltpu.async_copy(rows_v.at[b], out_slice(g), osem[b])
                pltpu.sync_copy(
                    idx_hbm.at[pl.ds(row0 + NBUF * C, C)], idx_v.at[b]
                )
                pltpu.make_async_copy(
                    rows_v.at[b], out_slice(g), osem[b]
                ).wait()
                pltpu.async_copy(lut_hbm.at[idx_v.at[b]], rows_v.at[b], gsem[b])
            return carry

        lax.fori_loop(0, ngroups - 1, group, 0)

        for b in range(NBUF):
            g = (ngroups - 1) * NBUF + b
            pltpu.make_async_copy(
                lut_hbm.at[idx_v.at[b]], rows_v.at[b], gsem[b]
            ).wait()
            scale_chunk(b)
            pltpu.async_copy(rows_v.at[b], out_slice(g), osem[b])
        for b in range(NBUF):
            g = (ngroups - 1) * NBUF + b
            pltpu.make_async_copy(rows_v.at[b], out_slice(g), osem[b]).wait()

    return k


def kernel(x, lut):
    B = x.shape[0] * x.shape[1]
    flat_idx = x.reshape(B)
    out128 = _make_kernel(B, D_MODEL, 640, 2, 8)(flat_idx, lut)
    return out128[:, :D_MODEL].reshape(x.shape[0], x.shape[1], D_MODEL)
